# baseline (speedup 1.0000x reference)
import jax
import jax.numpy as jnp
from jax import lax
from jax.experimental import pallas as pl
from jax.experimental.pallas import tpu as pltpu
from jax.experimental.pallas import tpu_sc as plsc

_V = 1000
_C = 128
_B = 1024
_L = 200
_LANES = 16
_NB = 13

_NW = 32
_BPW = _B // _NW


def _sc_body(x_hbm, mask_hbm, tab_hbm, out_hbm,
             idx_v, rows_v, mask_v, out_v, sem):
    wid = lax.axis_index("s") * 2 + lax.axis_index("c")
    iota = lax.broadcasted_iota(jnp.int32, (_LANES,), 0)

    def per_batch(i, carry):
        b = wid * _BPW + i
        pltpu.sync_copy(x_hbm.at[b], idx_v)
        pltpu.sync_copy(mask_hbm.at[b], mask_v.at[pl.ds(0, _L)])
        cp0 = pltpu.async_copy(tab_hbm.at[idx_v.at[pl.ds(0, 104)]],
                               rows_v.at[pl.ds(0, 104)], sem)
        cp1 = pltpu.async_copy(tab_hbm.at[idx_v.at[pl.ds(104, 96)]],
                               rows_v.at[pl.ds(104, 96)], sem)
        cp0.wait()
        cp1.wait()
        for lb in range(_NB):
            lvec = jnp.minimum(iota + (16 * lb), _L - 1)
            m = mask_v[pl.ds(16 * lb, _LANES)]
            valid = iota < (_L - 16 * lb)

            def per_c(c, cc, lvec=lvec, m=m, valid=valid, lb=lb):
                cvec = jnp.full((_LANES,), c, jnp.int32)
                vals = plsc.load_gather(rows_v, [lvec, cvec]) * m
                if lb < _NB - 1:
                    out_v[c, pl.ds(16 * lb, _LANES)] = vals
                else:
                    plsc.store_scatter(out_v, [cvec, lvec], vals, mask=valid)
                return cc

            lax.fori_loop(0, _C, per_c, 0)
        pltpu.sync_copy(out_v, out_hbm.at[b])
        return carry

    lax.fori_loop(0, _BPW, per_batch, 0)


def kernel(x, mask, emb_weight):
    x32 = x.astype(jnp.int32)
    mask2 = mask.reshape(_B, _L)
    mesh = plsc.VectorSubcoreMesh(core_axis_name="c", subcore_axis_name="s")
    run = pl.kernel(
        _sc_body,
        out_type=jax.ShapeDtypeStruct((_B, _C, _L), jnp.float32),
        mesh=mesh,
        compiler_params=pltpu.CompilerParams(
            needs_layout_passes=False, use_tc_tiling_on_sc=False),
        scratch_types=[
            pltpu.VMEM((_L,), jnp.int32),
            pltpu.VMEM((_L, _C), jnp.float32),
            pltpu.VMEM((208,), jnp.float32),
            pltpu.VMEM((_C, _L), jnp.float32),
            pltpu.SemaphoreType.DMA,
        ],
    )
    return run(x32, mask2, emb_weight)
